# Initial kernel scaffold; baseline (speedup 1.0000x reference)
import functools

import jax
import jax.numpy as jnp
from jax import lax
from jax.experimental import pallas as pl
from jax.experimental.pallas import tpu as pltpu
from jax.experimental.pallas import tpu_sc as plsc

T = 2048
D = 1024
E = 64
K = 8
F = 512
SF = 2816
N = T * K
TILE = 128
NPAD = N + E * TILE
NTILES = NPAD // TILE

NW = 32
TPW = T // NW
NPW = N // NW


_BT = 256


def _router_body(x_ref, gw_ref, logits_ref, tw_ref, ti_ref):
    x = x_ref[...]
    gw = gw_ref[...]
    logits = lax.dot_general(x, gw, (((1,), (1,)), ((), ())),
                             preferred_element_type=jnp.float32)
    logits_ref[...] = logits
    m = jnp.max(logits, axis=-1, keepdims=True)
    p = jnp.exp(logits - m)
    p = p / jnp.sum(p, axis=-1, keepdims=True)
    iota = lax.broadcasted_iota(jnp.int32, (_BT, E), 1)
    ws, ids = [], []
    for _ in range(K):
        mx = jnp.max(p, axis=-1, keepdims=True)
        sel = p == mx
        idx = jnp.min(jnp.where(sel, iota, E), axis=-1, keepdims=True)
        ws.append(mx)
        ids.append(idx)
        p = jnp.where(iota == idx, -1.0, p)
    tw = jnp.concatenate(ws, axis=1)
    tw_ref[...] = tw / jnp.sum(tw, axis=1, keepdims=True)
    ti_ref[...] = jnp.concatenate(ids, axis=1)


def _router(x, gate_w):
    return pl.pallas_call(
        _router_body,
        grid=(T // _BT,),
        in_specs=[
            pl.BlockSpec((_BT, D), lambda i: (i, 0)),
            pl.BlockSpec((E, D), lambda i: (0, 0)),
        ],
        out_specs=[
            pl.BlockSpec((_BT, E), lambda i: (i, 0)),
            pl.BlockSpec((_BT, K), lambda i: (i, 0)),
            pl.BlockSpec((_BT, K), lambda i: (i, 0)),
        ],
        out_shape=[
            jax.ShapeDtypeStruct((T, E), jnp.float32),
            jax.ShapeDtypeStruct((T, K), jnp.float32),
            jax.ShapeDtypeStruct((T, K), jnp.int32),
        ],
    )(x, gate_w)


_NC = N // TILE


def _dispatch_body(idx_ref, pos_ref, te_ref):
    idx = idx_ref[...]
    su = (lax.broadcasted_iota(jnp.int32, (_NC, _NC), 0)
          < lax.broadcasted_iota(jnp.int32, (_NC, _NC), 1)).astype(jnp.float32)
    sl = (lax.broadcasted_iota(jnp.int32, (_NC, _NC), 0)
          > lax.broadcasted_iota(jnp.int32, (_NC, _NC), 1)).astype(jnp.float32)
    totals = [jnp.sum((idx == e).astype(jnp.float32)) for e in range(E)]
    padded = [jnp.floor((t + 127.0) * (1.0 / 128.0)) * 128.0 for t in totals]
    pstart = []
    run = jnp.float32(0.0)
    for e in range(E):
        pstart.append(run)
        run = run + padded[e]
    tid = (lax.broadcasted_iota(jnp.float32, (2, TILE), 0) * TILE
           + lax.broadcasted_iota(jnp.float32, (2, TILE), 1))
    te = jnp.zeros((2, TILE), jnp.float32)
    for e in range(1, E):
        st = pstart[e] * (1.0 / 128.0)
        en = st + padded[e] * (1.0 / 128.0)
        te = te + jnp.float32(e) * ((tid >= st) & (tid < en)).astype(jnp.float32)
    te_ref[...] = te.astype(jnp.int32)
    posf = jnp.zeros((_NC, TILE), jnp.float32)
    for e in range(E):
        ind = (idx == e).astype(jnp.float32)
        excl = jnp.dot(ind, su, preferred_element_type=jnp.float32)
        cnt = jnp.sum(ind, axis=1, keepdims=True)
        cbase = jnp.dot(sl, cnt, preferred_element_type=jnp.float32)
        posf = posf + ind * (pstart[e] + cbase + excl)
    pos_ref[...] = posf.astype(jnp.int32)


def _dispatch(idx2):
    return pl.pallas_call(
        _dispatch_body,
        out_shape=[
            jax.ShapeDtypeStruct((_NC, TILE), jnp.int32),
            jax.ShapeDtypeStruct((2, TILE), jnp.int32),
        ],
    )(idx2)


_SC_MESH = plsc.VectorSubcoreMesh(core_axis_name="c", subcore_axis_name="s")


@functools.partial(
    pl.kernel,
    out_type=jax.ShapeDtypeStruct((NPAD, D), jnp.float32),
    mesh=_SC_MESH,
    scratch_types=[
        pltpu.VMEM((TPW, D), jnp.float32),
        pltpu.VMEM((8, 64), jnp.int32),
        pltpu.VMEM((K, TPW), jnp.int32),
        pltpu.SemaphoreType.DMA,
    ],
)
def _sc_gather(x_hbm, pos_hbm, xs_hbm, xv, posv, kidx, sem):
    w = lax.axis_index("s") * 2 + lax.axis_index("c")
    pltpu.sync_copy(x_hbm.at[pl.ds(w * TPW, TPW)], xv)
    pltpu.sync_copy(pos_hbm.at[pl.ds(w * 8, 8)], posv)
    lanes = lax.iota(jnp.int32, 16)
    for k in range(K):
        for g in range(TPW // 16):
            nl = (lanes + g * 16) * K + k
            row = lax.shift_right_logical(nl, 6)
            col = lax.bitwise_and(nl, 63)
            kidx[k, pl.ds(g * 16, 16)] = plsc.load_gather(posv, [row, col])
    for k in range(K):
        cp = pltpu.make_async_copy(xv, xs_hbm.at[kidx.at[k]], sem)
        cp.start()
        cp.wait()


def _ffn_body(te_ref, xs_ref, wg_ref, wu_ref, wd_ref, ys_ref):
    xt = xs_ref[...]
    g = jnp.dot(xt, wg_ref[0], preferred_element_type=jnp.float32)
    u = jnp.dot(xt, wu_ref[0], preferred_element_type=jnp.float32)
    a = g * jax.nn.sigmoid(g) * u
    ys_ref[...] = jnp.dot(a, wd_ref[0], preferred_element_type=jnp.float32)


def _ffn(te, xs, w_gate, w_up, w_down):
    grid_spec = pltpu.PrefetchScalarGridSpec(
        num_scalar_prefetch=1,
        grid=(NTILES,),
        in_specs=[
            pl.BlockSpec((TILE, D), lambda i, te: (i, 0)),
            pl.BlockSpec((1, D, F), lambda i, te: (te[i], 0, 0)),
            pl.BlockSpec((1, D, F), lambda i, te: (te[i], 0, 0)),
            pl.BlockSpec((1, F, D), lambda i, te: (te[i], 0, 0)),
        ],
        out_specs=pl.BlockSpec((TILE, D), lambda i, te: (i, 0)),
    )
    return pl.pallas_call(
        _ffn_body,
        grid_spec=grid_spec,
        out_shape=jax.ShapeDtypeStruct((NPAD, D), jnp.float32),
        compiler_params=pltpu.CompilerParams(
            dimension_semantics=("arbitrary",)),
    )(te, xs, w_gate, w_up, w_down)


@functools.partial(
    pl.kernel,
    out_type=jax.ShapeDtypeStruct((T, D), jnp.float32),
    mesh=_SC_MESH,
    scratch_types=[
        pltpu.VMEM((64, D), jnp.float32),
        pltpu.VMEM((8, 64), jnp.int32),
        pltpu.VMEM((8, 64), jnp.float32),
        pltpu.VMEM((8, D), jnp.float32),
        pltpu.SemaphoreType.DMA,
    ],
)
def _sc_combine(pos_hbm, tw_hbm, ys_hbm, moe_hbm, rows, posv, wv, acc, sem):
    w = lax.axis_index("s") * 2 + lax.axis_index("c")
    pltpu.sync_copy(pos_hbm.at[pl.ds(w * 8, 8)], posv)
    pltpu.sync_copy(tw_hbm.at[pl.ds(w * 8, 8)], wv)
    for c in range(8):
        cp = pltpu.make_async_copy(ys_hbm.at[posv.at[c]], rows, sem)
        cp.start()
        cp.wait()
        wts = [[wv[c, t * K + k] for k in range(K)] for t in range(8)]

        def dbody(d, _):
            off = pl.multiple_of(d * 16, 16)
            for t in range(8):
                a = wts[t][0] * rows[t * K, pl.ds(off, 16)]
                for k in range(1, K):
                    a = a + wts[t][k] * rows[t * K + k, pl.ds(off, 16)]
                acc[t, pl.ds(off, 16)] = a
            return 0

        lax.fori_loop(0, D // 16, dbody, 0)
        pltpu.sync_copy(acc, moe_hbm.at[pl.ds(w * TPW + c * 8, 8)])


def _shared_body(x_ref, moe_ref, sg_ref, su_ref, sd_ref, eg_ref, out_ref):
    x = x_ref[...]
    g = jnp.dot(x, sg_ref[...], preferred_element_type=jnp.float32)
    u = jnp.dot(x, su_ref[...], preferred_element_type=jnp.float32)
    a = g * jax.nn.sigmoid(g) * u
    sh = jnp.dot(a, sd_ref[...], preferred_element_type=jnp.float32)
    egate = jax.nn.sigmoid(jnp.dot(x, eg_ref[...],
                                   preferred_element_type=jnp.float32))
    out_ref[...] = moe_ref[...] + egate * sh


def _shared(x, moe, sh_gate_w, sh_up_w, sh_down_w, shared_expert_gate_w):
    return pl.pallas_call(
        _shared_body,
        grid=(T // _BT,),
        in_specs=[
            pl.BlockSpec((_BT, D), lambda i: (i, 0)),
            pl.BlockSpec((_BT, D), lambda i: (i, 0)),
            pl.BlockSpec((D, SF), lambda i: (0, 0)),
            pl.BlockSpec((D, SF), lambda i: (0, 0)),
            pl.BlockSpec((SF, D), lambda i: (0, 0)),
            pl.BlockSpec((D, 1), lambda i: (0, 0)),
        ],
        out_specs=pl.BlockSpec((_BT, D), lambda i: (i, 0)),
        out_shape=jax.ShapeDtypeStruct((T, D), jnp.float32),
    )(x, moe, sh_gate_w, sh_up_w, sh_down_w, shared_expert_gate_w)


def kernel(hidden_states, gate_w, W_gate, W_up, W_down,
           sh_gate_w, sh_up_w, sh_down_w, shared_expert_gate_w):
    x = hidden_states.reshape(T, D)
    logits, tw, ti = _router(x, gate_w)
    pos2, te2 = _dispatch(ti.reshape(_NC, TILE))
    pos_sc = pos2.reshape(N // 64, 64)
    xs = _sc_gather(x, pos_sc)
    ys = _ffn(te2.reshape(2 * TILE), xs, W_gate, W_up, W_down)
    moe = _sc_combine(pos_sc, tw.reshape(N // 64, 64), ys)
    out = _shared(x, moe, sh_gate_w, sh_up_w, sh_down_w, shared_expert_gate_w)
    return (out.reshape(1, T, D), logits)


# speedup vs baseline: 1.5722x; 1.5722x over previous
import functools

import jax
import jax.numpy as jnp
from jax import lax
from jax.experimental import pallas as pl
from jax.experimental.pallas import tpu as pltpu
from jax.experimental.pallas import tpu_sc as plsc

T = 2048
D = 1024
E = 64
K = 8
F = 512
SF = 2816
N = T * K
TILE = 128
NPAD = N + E * TILE
NTILES = NPAD // TILE

NW = 32
TPW = T // NW
NPW = N // NW


_BT = 256


def _router_body(x_ref, gw_ref, logits_ref, tw_ref, ti_ref):
    x = x_ref[...]
    gw = gw_ref[...]
    logits = lax.dot_general(x, gw, (((1,), (1,)), ((), ())),
                             preferred_element_type=jnp.float32)
    logits_ref[...] = logits
    m = jnp.max(logits, axis=-1, keepdims=True)
    p = jnp.exp(logits - m)
    p = p / jnp.sum(p, axis=-1, keepdims=True)
    iota = lax.broadcasted_iota(jnp.int32, (_BT, E), 1)
    ws, ids = [], []
    for _ in range(K):
        mx = jnp.max(p, axis=-1, keepdims=True)
        sel = p == mx
        idx = jnp.min(jnp.where(sel, iota, E), axis=-1, keepdims=True)
        ws.append(mx)
        ids.append(idx)
        p = jnp.where(iota == idx, -1.0, p)
    tw = jnp.concatenate(ws, axis=1)
    tw_ref[...] = tw / jnp.sum(tw, axis=1, keepdims=True)
    ti_ref[...] = jnp.concatenate(ids, axis=1)


def _router(x, gate_w):
    return pl.pallas_call(
        _router_body,
        grid=(T // _BT,),
        in_specs=[
            pl.BlockSpec((_BT, D), lambda i: (i, 0)),
            pl.BlockSpec((E, D), lambda i: (0, 0)),
        ],
        out_specs=[
            pl.BlockSpec((_BT, E), lambda i: (i, 0)),
            pl.BlockSpec((_BT, K), lambda i: (i, 0)),
            pl.BlockSpec((_BT, K), lambda i: (i, 0)),
        ],
        out_shape=[
            jax.ShapeDtypeStruct((T, E), jnp.float32),
            jax.ShapeDtypeStruct((T, K), jnp.float32),
            jax.ShapeDtypeStruct((T, K), jnp.int32),
        ],
    )(x, gate_w)


_NC = N // TILE


def _dispatch_body(idx_ref, pos_ref, te_ref):
    idx = idx_ref[...]
    su = (lax.broadcasted_iota(jnp.int32, (_NC, _NC), 0)
          < lax.broadcasted_iota(jnp.int32, (_NC, _NC), 1)).astype(jnp.float32)
    sl = (lax.broadcasted_iota(jnp.int32, (_NC, _NC), 0)
          > lax.broadcasted_iota(jnp.int32, (_NC, _NC), 1)).astype(jnp.float32)
    totals = [jnp.sum((idx == e).astype(jnp.float32)) for e in range(E)]
    padded = [jnp.floor((t + 127.0) * (1.0 / 128.0)) * 128.0 for t in totals]
    pstart = []
    run = jnp.float32(0.0)
    for e in range(E):
        pstart.append(run)
        run = run + padded[e]
    tid = (lax.broadcasted_iota(jnp.int32, (2, TILE), 0) * TILE
           + lax.broadcasted_iota(jnp.int32, (2, TILE), 1)).astype(jnp.float32)
    te = jnp.zeros((2, TILE), jnp.float32)
    for e in range(1, E):
        st = pstart[e] * (1.0 / 128.0)
        en = st + padded[e] * (1.0 / 128.0)
        te = te + jnp.float32(e) * ((tid >= st) & (tid < en)).astype(jnp.float32)
    te_ref[...] = te.astype(jnp.int32)
    posf = jnp.zeros((_NC, TILE), jnp.float32)
    for e in range(E):
        ind = (idx == e).astype(jnp.float32)
        excl = jnp.dot(ind, su, preferred_element_type=jnp.float32)
        cnt = jnp.sum(ind, axis=1, keepdims=True)
        cbase = jnp.dot(sl, cnt, preferred_element_type=jnp.float32)
        posf = posf + ind * (pstart[e] + cbase + excl)
    pos_ref[...] = posf.astype(jnp.int32)


def _dispatch(idx2):
    return pl.pallas_call(
        _dispatch_body,
        out_shape=[
            jax.ShapeDtypeStruct((_NC, TILE), jnp.int32),
            jax.ShapeDtypeStruct((2, TILE), jnp.int32),
        ],
    )(idx2)


@functools.cache
def _sc_gather_kernel():
    mesh = plsc.VectorSubcoreMesh(core_axis_name="c", subcore_axis_name="s")

    @functools.partial(
        pl.kernel,
        out_type=jax.ShapeDtypeStruct((NPAD, D), jnp.float32),
        mesh=mesh,
        scratch_types=[
            pltpu.VMEM((TPW, D), jnp.float32),
            pltpu.VMEM((8, 64), jnp.int32),
            pltpu.VMEM((K, TPW), jnp.int32),
            pltpu.SemaphoreType.DMA,
        ],
        compiler_params=pltpu.CompilerParams(needs_layout_passes=False),
    )
    def _sc_gather(x_hbm, pos_hbm, xs_hbm, xv, posv, kidx, sem):
        w = lax.axis_index("s") * 2 + lax.axis_index("c")
        pltpu.sync_copy(x_hbm.at[pl.ds(w * TPW, TPW)], xv)
        pltpu.sync_copy(pos_hbm.at[pl.ds(w * 8, 8)], posv)
        lanes = lax.iota(jnp.int32, 16)
        for k in range(K):
            for g in range(TPW // 16):
                nl = (lanes + g * 16) * K + k
                row = lax.shift_right_logical(nl, 6)
                col = lax.bitwise_and(nl, 63)
                kidx[k, pl.ds(g * 16, 16)] = plsc.load_gather(posv, [row, col])
        for k in range(K):
            cp = pltpu.make_async_copy(xv, xs_hbm.at[kidx.at[k]], sem)
            cp.start()
            cp.wait()

    return _sc_gather


def _ffn_body(te_ref, xs_ref, wg_ref, wu_ref, wd_ref, ys_ref):
    xt = xs_ref[...]
    g = jnp.dot(xt, wg_ref[0], preferred_element_type=jnp.float32)
    u = jnp.dot(xt, wu_ref[0], preferred_element_type=jnp.float32)
    a = g * jax.nn.sigmoid(g) * u
    ys_ref[...] = jnp.dot(a, wd_ref[0], preferred_element_type=jnp.float32)


def _ffn(te, xs, w_gate, w_up, w_down):
    grid_spec = pltpu.PrefetchScalarGridSpec(
        num_scalar_prefetch=1,
        grid=(NTILES,),
        in_specs=[
            pl.BlockSpec((TILE, D), lambda i, te: (i, 0)),
            pl.BlockSpec((1, D, F), lambda i, te: (te[i], 0, 0)),
            pl.BlockSpec((1, D, F), lambda i, te: (te[i], 0, 0)),
            pl.BlockSpec((1, F, D), lambda i, te: (te[i], 0, 0)),
        ],
        out_specs=pl.BlockSpec((TILE, D), lambda i, te: (i, 0)),
    )
    return pl.pallas_call(
        _ffn_body,
        grid_spec=grid_spec,
        out_shape=jax.ShapeDtypeStruct((NPAD, D), jnp.float32),
        compiler_params=pltpu.CompilerParams(
            dimension_semantics=("arbitrary",)),
    )(te, xs, w_gate, w_up, w_down)


@functools.cache
def _sc_combine_kernel():
    mesh = plsc.VectorSubcoreMesh(core_axis_name="c", subcore_axis_name="s")

    @functools.partial(
        pl.kernel,
        out_type=jax.ShapeDtypeStruct((T, D), jnp.float32),
        mesh=mesh,
        scratch_types=[
            pltpu.VMEM((64, D), jnp.float32),
            pltpu.VMEM((8, 64), jnp.int32),
            pltpu.VMEM((8, 64), jnp.float32),
            pltpu.VMEM((8, D), jnp.float32),
            pltpu.SemaphoreType.DMA,
        ],
        compiler_params=pltpu.CompilerParams(needs_layout_passes=False),
    )
    def _sc_combine(pos_hbm, tw_hbm, ys_hbm, moe_hbm, rows, posv, wv, acc,
                    sem):
        w = lax.axis_index("s") * 2 + lax.axis_index("c")
        pltpu.sync_copy(pos_hbm.at[pl.ds(w * 8, 8)], posv)
        pltpu.sync_copy(tw_hbm.at[pl.ds(w * 8, 8)], wv)
        for c in range(8):
            cp = pltpu.make_async_copy(ys_hbm.at[posv.at[c]], rows, sem)
            cp.start()
            cp.wait()
            wrow = [wv[c, pl.ds(g * 16, 16)] for g in range(4)]
            wts = [[wrow[(t * K + k) // 16][(t * K + k) % 16]
                    for k in range(K)] for t in range(8)]

            def dbody(d, _):
                off = pl.multiple_of(d * 16, 16)
                for t in range(8):
                    a = wts[t][0] * rows[t * K, pl.ds(off, 16)]
                    for k in range(1, K):
                        a = a + wts[t][k] * rows[t * K + k, pl.ds(off, 16)]
                    acc[t, pl.ds(off, 16)] = a
                return 0

            lax.fori_loop(0, D // 16, dbody, 0)
            pltpu.sync_copy(acc, moe_hbm.at[pl.ds(w * TPW + c * 8, 8)])

    return _sc_combine


def _shared_body(x_ref, moe_ref, sg_ref, su_ref, sd_ref, eg_ref, out_ref):
    x = x_ref[...]
    g = jnp.dot(x, sg_ref[...], preferred_element_type=jnp.float32)
    u = jnp.dot(x, su_ref[...], preferred_element_type=jnp.float32)
    a = g * jax.nn.sigmoid(g) * u
    sh = jnp.dot(a, sd_ref[...], preferred_element_type=jnp.float32)
    egate = jax.nn.sigmoid(jnp.dot(x, eg_ref[...],
                                   preferred_element_type=jnp.float32))
    out_ref[...] = moe_ref[...] + egate * sh


def _shared(x, moe, sh_gate_w, sh_up_w, sh_down_w, shared_expert_gate_w):
    return pl.pallas_call(
        _shared_body,
        grid=(T // _BT,),
        in_specs=[
            pl.BlockSpec((_BT, D), lambda i: (i, 0)),
            pl.BlockSpec((_BT, D), lambda i: (i, 0)),
            pl.BlockSpec((D, SF), lambda i: (0, 0)),
            pl.BlockSpec((D, SF), lambda i: (0, 0)),
            pl.BlockSpec((SF, D), lambda i: (0, 0)),
            pl.BlockSpec((D, 1), lambda i: (0, 0)),
        ],
        out_specs=pl.BlockSpec((_BT, D), lambda i: (i, 0)),
        out_shape=jax.ShapeDtypeStruct((T, D), jnp.float32),
    )(x, moe, sh_gate_w, sh_up_w, sh_down_w, shared_expert_gate_w)


def kernel(hidden_states, gate_w, W_gate, W_up, W_down,
           sh_gate_w, sh_up_w, sh_down_w, shared_expert_gate_w):
    x = hidden_states.reshape(T, D)
    logits, tw, ti = _router(x, gate_w)
    pos2, te2 = _dispatch(ti.reshape(_NC, TILE))
    pos_sc = pos2.reshape(N // 64, 64)
    xs = _sc_gather_kernel()(x, pos_sc)
    ys = _ffn(te2.reshape(2 * TILE), xs, W_gate, W_up, W_down)
    moe = _sc_combine_kernel()(pos_sc, tw.reshape(N // 64, 64), ys)
    out = _shared(x, moe, sh_gate_w, sh_up_w, sh_down_w, shared_expert_gate_w)
    return (out.reshape(1, T, D), logits)


# speedup vs baseline: 1.5891x; 1.0108x over previous
import functools

import jax
import jax.numpy as jnp
from jax import lax
from jax.experimental import pallas as pl
from jax.experimental.pallas import tpu as pltpu
from jax.experimental.pallas import tpu_sc as plsc

T = 2048
D = 1024
E = 64
K = 8
F = 512
SF = 2816
N = T * K
TILE = 128
NPAD = N + E * TILE
NTILES = NPAD // TILE

NW = 32
TPW = T // NW
NPW = N // NW


_BT = 256


def _router_body(x_ref, gw_ref, logits_ref, tw_ref, ti_ref):
    x = x_ref[...]
    gw = gw_ref[...]
    logits = lax.dot_general(x, gw, (((1,), (1,)), ((), ())),
                             preferred_element_type=jnp.float32)
    logits_ref[...] = logits
    m = jnp.max(logits, axis=-1, keepdims=True)
    p = jnp.exp(logits - m)
    p = p / jnp.sum(p, axis=-1, keepdims=True)
    iota = lax.broadcasted_iota(jnp.int32, (_BT, E), 1)
    ws, ids = [], []
    for _ in range(K):
        mx = jnp.max(p, axis=-1, keepdims=True)
        sel = p == mx
        idx = jnp.min(jnp.where(sel, iota, E), axis=-1, keepdims=True)
        ws.append(mx)
        ids.append(idx)
        p = jnp.where(iota == idx, -1.0, p)
    tw = jnp.concatenate(ws, axis=1)
    tw_ref[...] = tw / jnp.sum(tw, axis=1, keepdims=True)
    ti_ref[...] = jnp.concatenate(ids, axis=1)


def _router(x, gate_w):
    return pl.pallas_call(
        _router_body,
        grid=(T // _BT,),
        in_specs=[
            pl.BlockSpec((_BT, D), lambda i: (i, 0)),
            pl.BlockSpec((E, D), lambda i: (0, 0)),
        ],
        out_specs=[
            pl.BlockSpec((_BT, E), lambda i: (i, 0)),
            pl.BlockSpec((_BT, K), lambda i: (i, 0)),
            pl.BlockSpec((_BT, K), lambda i: (i, 0)),
        ],
        out_shape=[
            jax.ShapeDtypeStruct((T, E), jnp.float32),
            jax.ShapeDtypeStruct((T, K), jnp.float32),
            jax.ShapeDtypeStruct((T, K), jnp.int32),
        ],
    )(x, gate_w)


_NC = N // TILE


def _dispatch_body(idx_ref, pos_ref, te_ref):
    idx = idx_ref[...]
    su = (lax.broadcasted_iota(jnp.int32, (_NC, _NC), 0)
          < lax.broadcasted_iota(jnp.int32, (_NC, _NC), 1)).astype(jnp.float32)
    sl = (lax.broadcasted_iota(jnp.int32, (_NC, _NC), 0)
          > lax.broadcasted_iota(jnp.int32, (_NC, _NC), 1)).astype(jnp.float32)
    totals = [jnp.sum((idx == e).astype(jnp.float32)) for e in range(E)]
    padded = [jnp.floor((t + 127.0) * (1.0 / 128.0)) * 128.0 for t in totals]
    pstart = []
    run = jnp.float32(0.0)
    for e in range(E):
        pstart.append(run)
        run = run + padded[e]
    ntiles = run * (1.0 / 128.0)
    lastu = jnp.float32(0.0)
    for e in range(1, E):
        lastu = lastu + jnp.float32(e) * (
            (padded[e] > 0.0) & (pstart[e] + padded[e] == run)
        ).astype(jnp.float32)
    tid = (lax.broadcasted_iota(jnp.int32, (2, TILE), 0) * TILE
           + lax.broadcasted_iota(jnp.int32, (2, TILE), 1)).astype(jnp.float32)
    te = jnp.zeros((2, TILE), jnp.float32)
    for e in range(1, E):
        st = pstart[e] * (1.0 / 128.0)
        en = st + padded[e] * (1.0 / 128.0)
        te = te + jnp.float32(e) * ((tid >= st) & (tid < en)).astype(jnp.float32)
    te = te + lastu * ((tid >= ntiles) & (tid < jnp.float32(NTILES))
                       ).astype(jnp.float32)
    te = te + ntiles * (tid == jnp.float32(NTILES)).astype(jnp.float32)
    te_ref[...] = te.astype(jnp.int32)
    posf = jnp.zeros((_NC, TILE), jnp.float32)
    for e in range(E):
        ind = (idx == e).astype(jnp.float32)
        excl = jnp.dot(ind, su, preferred_element_type=jnp.float32)
        cnt = jnp.sum(ind, axis=1, keepdims=True)
        cbase = jnp.dot(sl, cnt, preferred_element_type=jnp.float32)
        posf = posf + ind * (pstart[e] + cbase + excl)
    pos_ref[...] = posf.astype(jnp.int32)


def _dispatch(idx2):
    return pl.pallas_call(
        _dispatch_body,
        out_shape=[
            jax.ShapeDtypeStruct((_NC, TILE), jnp.int32),
            jax.ShapeDtypeStruct((2, TILE), jnp.int32),
        ],
    )(idx2)


@functools.cache
def _sc_gather_kernel():
    mesh = plsc.VectorSubcoreMesh(core_axis_name="c", subcore_axis_name="s")

    @functools.partial(
        pl.kernel,
        out_type=jax.ShapeDtypeStruct((NPAD, D), jnp.float32),
        mesh=mesh,
        scratch_types=[
            pltpu.VMEM((TPW, D), jnp.float32),
            pltpu.VMEM((8, 64), jnp.int32),
            pltpu.VMEM((K, TPW), jnp.int32),
            pltpu.SemaphoreType.DMA,
        ],
        compiler_params=pltpu.CompilerParams(needs_layout_passes=False),
    )
    def _sc_gather(x_hbm, pos_hbm, xs_hbm, xv, posv, kidx, sem):
        w = lax.axis_index("s") * 2 + lax.axis_index("c")
        pltpu.sync_copy(x_hbm.at[pl.ds(w * TPW, TPW)], xv)
        pltpu.sync_copy(pos_hbm.at[pl.ds(w * 8, 8)], posv)
        lanes = lax.iota(jnp.int32, 16)
        for k in range(K):
            for g in range(TPW // 16):
                nl = (lanes + g * 16) * K + k
                row = lax.shift_right_logical(nl, 6)
                col = lax.bitwise_and(nl, 63)
                kidx[k, pl.ds(g * 16, 16)] = plsc.load_gather(posv, [row, col])
        cps = [pltpu.make_async_copy(xv, xs_hbm.at[kidx.at[k]], sem)
               for k in range(K)]
        for cp in cps:
            cp.start()
        for cp in cps:
            cp.wait()

    return _sc_gather


def _ffn_body(te_ref, xs_ref, wg_ref, wu_ref, wd_ref, ys_ref):
    @pl.when(pl.program_id(0) < te_ref[NTILES])
    def _():
        xt = xs_ref[...]
        g = jnp.dot(xt, wg_ref[0], preferred_element_type=jnp.float32)
        u = jnp.dot(xt, wu_ref[0], preferred_element_type=jnp.float32)
        a = g * jax.nn.sigmoid(g) * u
        ys_ref[...] = jnp.dot(a, wd_ref[0], preferred_element_type=jnp.float32)


def _ffn(te, xs, w_gate, w_up, w_down):
    grid_spec = pltpu.PrefetchScalarGridSpec(
        num_scalar_prefetch=1,
        grid=(NTILES,),
        in_specs=[
            pl.BlockSpec((TILE, D), lambda i, te: (i, 0)),
            pl.BlockSpec((1, D, F), lambda i, te: (te[i], 0, 0)),
            pl.BlockSpec((1, D, F), lambda i, te: (te[i], 0, 0)),
            pl.BlockSpec((1, F, D), lambda i, te: (te[i], 0, 0)),
        ],
        out_specs=pl.BlockSpec((TILE, D), lambda i, te: (i, 0)),
    )
    return pl.pallas_call(
        _ffn_body,
        grid_spec=grid_spec,
        out_shape=jax.ShapeDtypeStruct((NPAD, D), jnp.float32),
        compiler_params=pltpu.CompilerParams(
            dimension_semantics=("arbitrary",)),
    )(te, xs, w_gate, w_up, w_down)


@functools.cache
def _sc_combine_kernel():
    mesh = plsc.VectorSubcoreMesh(core_axis_name="c", subcore_axis_name="s")
    nch = 16

    @functools.partial(
        pl.kernel,
        out_type=jax.ShapeDtypeStruct((T, D), jnp.float32),
        mesh=mesh,
        scratch_types=[
            pltpu.VMEM((2, 32, D), jnp.float32),
            pltpu.VMEM((2, 4, D), jnp.float32),
            pltpu.VMEM((16, 32), jnp.int32),
            pltpu.VMEM((16, 32), jnp.float32),
            pltpu.SemaphoreType.DMA,
            pltpu.SemaphoreType.DMA,
            pltpu.SemaphoreType.DMA,
            pltpu.SemaphoreType.DMA,
        ],
        compiler_params=pltpu.CompilerParams(needs_layout_passes=False),
    )
    def _sc_combine(pos_hbm, tw_hbm, ys_hbm, sh_hbm, out_hbm,
                    rows, acc, posv, wv, sem0, sem1, sem2, sem3):
        w = lax.axis_index("s") * 2 + lax.axis_index("c")
        pltpu.sync_copy(pos_hbm.at[pl.ds(w * 16, 16)], posv)
        pltpu.sync_copy(tw_hbm.at[pl.ds(w * 16, 16)], wv)
        rsem = [sem0, sem1]
        asem = [sem2, sem3]

        def start(i, b):
            pltpu.make_async_copy(
                ys_hbm.at[posv.at[i]], rows.at[b], rsem[b]).start()
            pltpu.make_async_copy(
                sh_hbm.at[pl.ds(w * TPW + i * 4, 4)], acc.at[b],
                asem[b]).start()

        start(0, 0)
        for i in range(nch):
            b = i % 2
            if i + 1 < nch:
                start(i + 1, 1 - b)
            pltpu.make_async_copy(
                ys_hbm.at[posv.at[i]], rows.at[b], rsem[b]).wait()
            pltpu.make_async_copy(
                sh_hbm.at[pl.ds(w * TPW + i * 4, 4)], acc.at[b],
                asem[b]).wait()
            wrow = [wv[i, pl.ds(g * 16, 16)] for g in range(2)]
            wts = [[wrow[(t * K + k) // 16][(t * K + k) % 16]
                    for k in range(K)] for t in range(4)]

            def dbody(d, _, b=b, wts=wts):
                off = pl.multiple_of(d * 16, 16)
                for t in range(4):
                    a = acc[b, t, pl.ds(off, 16)]
                    for k in range(K):
                        a = a + wts[t][k] * rows[b, t * K + k, pl.ds(off, 16)]
                    acc[b, t, pl.ds(off, 16)] = a
                return 0

            lax.fori_loop(0, D // 16, dbody, 0)
            pltpu.sync_copy(acc.at[b],
                            out_hbm.at[pl.ds(w * TPW + i * 4, 4)])

    return _sc_combine


def _shared_body(x_ref, sg_ref, su_ref, sd_ref, eg_ref, out_ref):
    x = x_ref[...]
    g = jnp.dot(x, sg_ref[...], preferred_element_type=jnp.float32)
    u = jnp.dot(x, su_ref[...], preferred_element_type=jnp.float32)
    a = g * jax.nn.sigmoid(g) * u
    sh = jnp.dot(a, sd_ref[...], preferred_element_type=jnp.float32)
    egate = jax.nn.sigmoid(jnp.dot(x, eg_ref[...],
                                   preferred_element_type=jnp.float32))
    out_ref[...] = egate * sh


def _shared(x, sh_gate_w, sh_up_w, sh_down_w, shared_expert_gate_w):
    return pl.pallas_call(
        _shared_body,
        grid=(T // _BT,),
        in_specs=[
            pl.BlockSpec((_BT, D), lambda i: (i, 0)),
            pl.BlockSpec((D, SF), lambda i: (0, 0)),
            pl.BlockSpec((D, SF), lambda i: (0, 0)),
            pl.BlockSpec((SF, D), lambda i: (0, 0)),
            pl.BlockSpec((D, 1), lambda i: (0, 0)),
        ],
        out_specs=pl.BlockSpec((_BT, D), lambda i: (i, 0)),
        out_shape=jax.ShapeDtypeStruct((T, D), jnp.float32),
    )(x, sh_gate_w, sh_up_w, sh_down_w, shared_expert_gate_w)


def kernel(hidden_states, gate_w, W_gate, W_up, W_down,
           sh_gate_w, sh_up_w, sh_down_w, shared_expert_gate_w):
    x = hidden_states.reshape(T, D)
    logits, tw, ti = _router(x, gate_w)
    shared = _shared(x, sh_gate_w, sh_up_w, sh_down_w, shared_expert_gate_w)
    pos2, te2 = _dispatch(ti.reshape(_NC, TILE))
    xs = _sc_gather_kernel()(x, pos2.reshape(N // 64, 64))
    ys = _ffn(te2.reshape(2 * TILE), xs, W_gate, W_up, W_down)
    out = _sc_combine_kernel()(pos2.reshape(N // 32, 32),
                               tw.reshape(N // 32, 32), ys, shared)
    return (out.reshape(1, T, D), logits)
